# Initial kernel scaffold; baseline (speedup 1.0000x reference)
import jax
import jax.numpy as jnp
from jax.experimental import pallas as pl


def kernel(mu, distances, distance_vector, neighbors, f_ij, neighbor_mask, W1, b1, W2, b2, We1, be1, We2, be2):
    raise NotImplementedError("write your pallas kernel here")


# speedup vs baseline: 9.2889x; 9.2889x over previous
import functools

import jax
import jax.numpy as jnp
from jax import lax
from jax.experimental import pallas as pl
from jax.experimental.pallas import tpu as pltpu
from jax.experimental.pallas import tpu_sc as plsc

_CUTOFF = 5.0
_LOG2 = 0.6931471805599453


def _sc_gather(table, idx2d):
    R, D = table.shape
    n_rows, CH = idx2d.shape
    E = n_rows * CH
    NW = 32
    rows_per_w = n_rows // NW

    mesh = plsc.VectorSubcoreMesh(core_axis_name="c", subcore_axis_name="s")

    @functools.partial(
        pl.kernel,
        mesh=mesh,
        out_type=jax.ShapeDtypeStruct((E, D), jnp.float32),
        scratch_types=[
            pltpu.VMEM((rows_per_w, CH), jnp.int32),
            pltpu.VMEM((CH, D), jnp.float32),
            pltpu.VMEM((CH, D), jnp.float32),
            pltpu.SemaphoreType.DMA,
            pltpu.SemaphoreType.DMA,
            pltpu.SemaphoreType.DMA,
            pltpu.SemaphoreType.DMA,
        ],
    )
    def k(table_hbm, idx_hbm, out_hbm, idx_v, rows0, rows1, g0, g1, p0, p1):
        wid = lax.axis_index("s") * 2 + lax.axis_index("c")
        row0 = wid * rows_per_w
        pltpu.sync_copy(idx_hbm.at[pl.ds(row0, rows_per_w)], idx_v)
        bufs = (rows0, rows1)
        gsems = (g0, g1)
        psems = (p0, p1)
        gh = [None, None]
        ph = [None, None]
        gh[0] = pltpu.async_copy(table_hbm.at[idx_v.at[0]], bufs[0], gsems[0])
        for c in range(rows_per_w):
            s = c % 2
            n = (c + 1) % 2
            if c + 1 < rows_per_w:
                if ph[n] is not None:
                    ph[n].wait()
                gh[n] = pltpu.async_copy(
                    table_hbm.at[idx_v.at[c + 1]], bufs[n], gsems[n])
            gh[s].wait()
            ph[s] = pltpu.async_copy(
                bufs[s], out_hbm.at[pl.ds((row0 + c) * CH, CH)], psems[s])
        ph[(rows_per_w - 1) % 2].wait()
        if rows_per_w > 1:
            ph[rows_per_w % 2].wait()

    return k(table, idx2d)


def _ssp(x):
    return jnp.maximum(x, 0.0) + jnp.log1p(jnp.exp(-jnp.abs(x))) - _LOG2


def _tc_body(muj_ref, mui_ref, fij_ref, aux_ref,
             we1_ref, be1_ref, we2_ref, be2_ref,
             w1_ref, b1_ref, w2_ref, b2_ref, out_ref, *, n_nbr):
    EB, D = muj_ref.shape
    TA = mui_ref.shape[0]
    F = D // 3

    fj = fij_ref[...]
    h = _ssp(jnp.dot(fj, we1_ref[...], preferred_element_type=jnp.float32)
             + be1_ref[...])
    rad = (jnp.dot(h, we2_ref[...], preferred_element_type=jnp.float32)
           + be2_ref[...])

    aux = aux_ref[...]
    d = aux[:, 3:4]
    msk = aux[:, 4:5]
    cm = (d + 1e-7 < _CUTOFF).astype(jnp.float32)
    dm = d * (cm * (1.0 / _CUTOFF))
    cut = jnp.exp(1.0 - 1.0 / (1.0 - dm * dm)) * cm
    d2 = d * d
    rad = rad * (cut * msk / (d2 * d2 * d))

    muj = muj_ref[...]
    mui = mui_ref[...]
    c3d2 = 3.0 * d2
    t = jnp.zeros((EB, F), jnp.float32)
    for x in range(3):
        mjx = muj[:, x * F:(x + 1) * F]
        mix = mui[:, x * F:(x + 1) * F]
        mib = jnp.broadcast_to(
            mix[:, None, :], (TA, n_nbr, F)).reshape(EB, F)
        dvx = aux[:, x:x + 1]
        t = t + (mib * mjx) * c3d2 - (mib + mjx) * dvx

    vpre = (t * rad).reshape(TA, n_nbr, F).sum(axis=1)

    v = _ssp(jnp.dot(vpre, w1_ref[...], preferred_element_type=jnp.float32)
             + b1_ref[...])
    out_ref[...] = (jnp.dot(v, w2_ref[...], preferred_element_type=jnp.float32)
                    + b2_ref[...])


def _tc_call(muj, mu_r, fij, aux, We1, be1, We2, be2, W1, b1, W2, b2, n_nbr):
    R, D = mu_r.shape
    E = muj.shape[0]
    G = fij.shape[1]
    AF = W2.shape[1]
    TA = 128
    EB = TA * n_nbr
    grid = (R // TA,)

    full = lambda a: pl.BlockSpec(a.shape, lambda i: (0, 0))
    return pl.pallas_call(
        functools.partial(_tc_body, n_nbr=n_nbr),
        grid=grid,
        in_specs=[
            pl.BlockSpec((EB, D), lambda i: (i, 0)),
            pl.BlockSpec((TA, D), lambda i: (i, 0)),
            pl.BlockSpec((EB, G), lambda i: (i, 0)),
            pl.BlockSpec((EB, 8), lambda i: (i, 0)),
            full(We1), full(be1), full(We2), full(be2),
            full(W1), full(b1), full(W2), full(b2),
        ],
        out_specs=pl.BlockSpec((TA, AF), lambda i: (i, 0)),
        out_shape=jax.ShapeDtypeStruct((R, AF), jnp.float32),
    )(muj, mu_r, fij, aux, We1, be1, We2, be2, W1, b1, W2, b2)


def kernel(mu, distances, distance_vector, neighbors, f_ij, neighbor_mask,
           W1, b1, W2, b2, We1, be1, We2, be2):
    B, A, F, X = mu.shape
    N = distances.shape[-1]
    G = f_ij.shape[-1]
    E = B * A * N

    mu_r = mu.transpose(0, 1, 3, 2).reshape(B * A, X * F)
    idx = (neighbors.astype(jnp.int32)
           + (jnp.arange(B, dtype=jnp.int32) * A)[:, None, None])
    muj = _sc_gather(mu_r, idx.reshape(E // 128, 128))

    aux = jnp.concatenate([
        distance_vector.reshape(E, X).astype(jnp.float32),
        distances.reshape(E, 1).astype(jnp.float32),
        neighbor_mask.reshape(E, 1).astype(jnp.float32),
        jnp.zeros((E, 3), jnp.float32),
    ], axis=1)
    fij = f_ij.reshape(E, G)

    out = _tc_call(muj, mu_r, fij, aux,
                   We1, be1.reshape(1, -1), We2, be2.reshape(1, -1),
                   W1, b1.reshape(1, -1), W2, b2.reshape(1, -1), N)
    return out.reshape(B, A, -1)
